# Initial kernel scaffold; baseline (speedup 1.0000x reference)
import jax
import jax.numpy as jnp
from jax.experimental import pallas as pl


def kernel(h, edge_index, W_fc, W_attn):
    raise NotImplementedError("write your pallas kernel here")


# speedup vs baseline: 1.0804x; 1.0804x over previous
import functools

import jax
import jax.numpy as jnp
from jax.experimental import pallas as pl

N = 10000
E = 320000
DIM = 128
ROW_BLK = 1000


def _fc_body(h_ref, wt_ref, a_ref, z_ref, pq_ref):
    z = h_ref[...] @ wt_ref[...]
    z_ref[...] = z
    pq_ref[...] = z @ a_ref[...]


def _fc_call(h, W_fc, A_pad):
    grid = (N // ROW_BLK,)
    return pl.pallas_call(
        _fc_body,
        grid=grid,
        in_specs=[
            pl.BlockSpec((ROW_BLK, DIM), lambda i: (i, 0)),
            pl.BlockSpec((DIM, DIM), lambda i: (0, 0)),
            pl.BlockSpec((DIM, DIM), lambda i: (0, 0)),
        ],
        out_specs=[
            pl.BlockSpec((ROW_BLK, DIM), lambda i: (i, 0)),
            pl.BlockSpec((ROW_BLK, DIM), lambda i: (i, 0)),
        ],
        out_shape=[
            jax.ShapeDtypeStruct((N, DIM), jnp.float32),
            jax.ShapeDtypeStruct((N, DIM), jnp.float32),
        ],
    )(h, W_fc, A_pad)


def kernel(h, edge_index, W_fc, W_attn):
    a_src = W_attn[0, :DIM]
    a_dst = W_attn[0, DIM:]
    A_pad = jnp.zeros((DIM, DIM), jnp.float32)
    A_pad = A_pad.at[:, 0].set(a_src).at[:, 1].set(a_dst)

    z, pq = _fc_call(h, W_fc.T, A_pad)
    p = pq[:, 0]
    q = pq[:, 1]

    src = edge_index[0]
    dst = edge_index[1]
    e = p[src] + q[dst]
    e = jnp.where(e > 0, e, 0.01 * e)
    ex = jnp.exp(e)
    s = jax.ops.segment_sum(ex, dst, num_segments=N)
    alpha = ex / jnp.maximum(s[dst], 1e-16)
    h_out = jax.ops.segment_sum(alpha[:, None] * z[src], dst, num_segments=N)
    h_out = jnp.where(h_out > 0, h_out, jnp.expm1(h_out))
    return h_out


# speedup vs baseline: 15.7252x; 14.5544x over previous
import functools

import jax
import jax.numpy as jnp
from jax import lax
from jax.experimental import pallas as pl
from jax.experimental.pallas import tpu as pltpu
from jax.experimental.pallas import tpu_sc as plsc

N = 10000
E = 320000
DIM = 128
WID = 144
ROW_BLK = 1000

NC = 2
NS = 16
NW = NC * NS
EPW = E // NW
CH = 80
NCH = EPW // CH
GR = CH // 16
NR = DIM // 16 + 1
ZR = 125
RPT = N // NS


def _fc_body(h_ref, wt_ref, a_ref, z_ref, pq_ref):
    z = h_ref[...] @ wt_ref[...]
    blk = z.shape[0]
    z_ref[...] = jnp.concatenate(
        [z, jnp.ones((blk, 1), jnp.float32), jnp.zeros((blk, WID - DIM - 1), jnp.float32)],
        axis=1)
    pq_ref[...] = z @ a_ref[...]


def _fc_call(h, W_fcT, A_pad):
    return pl.pallas_call(
        _fc_body,
        grid=(N // ROW_BLK,),
        in_specs=[
            pl.BlockSpec((ROW_BLK, DIM), lambda i: (i, 0)),
            pl.BlockSpec((DIM, DIM), lambda i: (0, 0)),
            pl.BlockSpec((DIM, DIM), lambda i: (0, 0)),
        ],
        out_specs=[
            pl.BlockSpec((ROW_BLK, WID), lambda i: (i, 0)),
            pl.BlockSpec((ROW_BLK, DIM), lambda i: (i, 0)),
        ],
        out_shape=[
            jax.ShapeDtypeStruct((N, WID), jnp.float32),
            jax.ShapeDtypeStruct((N, DIM), jnp.float32),
        ],
    )(h, W_fcT, A_pad)


def _edge_body(z_ref, src_ref, dst_ref, p_ref, q_ref, out_ref,
               src_v, dst_v, ex_v, pg_v, qg_v, rows_v, zrow_v, out_sh,
               sem_r, sem_p, sem_q):
    cid = lax.axis_index("c")
    sid = lax.axis_index("s")
    wid = sid * NC + cid

    def _zr(i, carry):
        r = i // NR
        j = i % NR
        zrow_v[r, pl.ds(j * 16, 16)] = jnp.zeros((16,), jnp.float32)
        return carry
    lax.fori_loop(0, ZR * NR, _zr, 0)
    for cz in range(RPT // ZR):
        pltpu.sync_copy(zrow_v, out_sh.at[pl.ds(sid * RPT + cz * ZR, ZR)])
    plsc.subcore_barrier()

    ebase = wid * EPW

    def _chunk(c, carry):
        off = ebase + c * CH
        pltpu.sync_copy(src_ref.at[pl.ds(off, CH)], src_v)
        pltpu.sync_copy(dst_ref.at[pl.ds(off, CH)], dst_v)
        cp_r = pltpu.async_copy(z_ref.at[src_v], rows_v, sem_r)
        cp_p = pltpu.async_copy(p_ref.at[src_v], pg_v, sem_p)
        cp_q = pltpu.async_copy(q_ref.at[dst_v], qg_v, sem_q)
        cp_p.wait()
        cp_q.wait()
        for g in range(GR):
            e = pg_v[pl.ds(g * 16, 16)] + qg_v[pl.ds(g * 16, 16)]
            e = jnp.where(e > 0.0, e, 0.01 * e)
            ex_v[pl.ds(g * 16, 16)] = jnp.exp(e)
        cp_r.wait()
        for g in range(GR):
            ve = ex_v[pl.ds(g * 16, 16)]
            for kk in range(16):
                k = g * 16 + kk
                av = lax.gather(
                    ve, jnp.full((16, 1), kk, jnp.int32),
                    lax.GatherDimensionNumbers(
                        offset_dims=(), collapsed_slice_dims=(0,),
                        start_index_map=(0,)),
                    slice_sizes=(1,),
                    mode=lax.GatherScatterMode.PROMISE_IN_BOUNDS)
                for j in range(NR):
                    rows_v[k, pl.ds(j * 16, 16)] = rows_v[k, pl.ds(j * 16, 16)] * av
        pltpu.sync_copy(rows_v, out_sh.at[dst_v], add=True)
        return carry

    lax.fori_loop(0, NCH, _chunk, 0)
    plsc.subcore_barrier()

    pltpu.sync_copy(out_sh.at[pl.ds(sid * RPT, RPT)],
                    out_ref.at[cid, pl.ds(sid * RPT, RPT)])


@functools.partial(
    pl.kernel,
    out_type=jax.ShapeDtypeStruct((NC, N, WID), jnp.float32),
    mesh=plsc.VectorSubcoreMesh(
        core_axis_name="c", subcore_axis_name="s", num_cores=NC, num_subcores=NS),
    scratch_types=[
        pltpu.VMEM((CH,), jnp.int32),
        pltpu.VMEM((CH,), jnp.int32),
        pltpu.VMEM((CH,), jnp.float32),
        pltpu.VMEM((CH,), jnp.float32),
        pltpu.VMEM((CH,), jnp.float32),
        pltpu.VMEM((CH, WID), jnp.float32),
        pltpu.VMEM((ZR, WID), jnp.float32),
        pltpu.VMEM_SHARED((N, WID), jnp.float32),
        pltpu.SemaphoreType.DMA,
        pltpu.SemaphoreType.DMA,
        pltpu.SemaphoreType.DMA,
    ],
    compiler_params=pltpu.CompilerParams(
        needs_layout_passes=False, use_tc_tiling_on_sc=False),
)
def _edge_call(z_ref, src_ref, dst_ref, p_ref, q_ref, out_ref,
               src_v, dst_v, ex_v, pg_v, qg_v, rows_v, zrow_v, out_sh,
               sem_r, sem_p, sem_q):
    _edge_body(z_ref, src_ref, dst_ref, p_ref, q_ref, out_ref,
               src_v, dst_v, ex_v, pg_v, qg_v, rows_v, zrow_v, out_sh,
               sem_r, sem_p, sem_q)


def _merge_body(parts_ref, out_ref):
    v = parts_ref[...]
    num = v[0, :, :DIM] + v[1, :, :DIM]
    den = v[0, :, DIM:DIM + 1] + v[1, :, DIM:DIM + 1]
    hval = num / jnp.maximum(den, 1e-16)
    out_ref[...] = jnp.where(hval > 0.0, hval, jnp.exp(hval) - 1.0)


def _merge_call(parts):
    return pl.pallas_call(
        _merge_body,
        grid=(N // ROW_BLK,),
        in_specs=[pl.BlockSpec((NC, ROW_BLK, WID), lambda i: (0, i, 0))],
        out_specs=pl.BlockSpec((ROW_BLK, DIM), lambda i: (i, 0)),
        out_shape=jax.ShapeDtypeStruct((N, DIM), jnp.float32),
    )(parts)


def kernel(h, edge_index, W_fc, W_attn):
    a_src = W_attn[0, :DIM]
    a_dst = W_attn[0, DIM:]
    A_pad = jnp.zeros((DIM, DIM), jnp.float32)
    A_pad = A_pad.at[:, 0].set(a_src).at[:, 1].set(a_dst)

    z_aug, pq = _fc_call(h, W_fc.T, A_pad)
    p = pq[:, 0]
    q = pq[:, 1]
    src = edge_index[0]
    dst = edge_index[1]

    parts = _edge_call(z_aug, src, dst, p, q)
    return _merge_call(parts)


# speedup vs baseline: 29.4934x; 1.8755x over previous
import functools

import jax
import jax.numpy as jnp
from jax import lax
from jax.experimental import pallas as pl
from jax.experimental.pallas import tpu as pltpu
from jax.experimental.pallas import tpu_sc as plsc

N = 10000
E = 320000
DIM = 128
WID = 144
ROW_BLK = 1000

NC = 2
NS = 16
NW = NC * NS
EPW = E // NW
CH = 80
NCH = EPW // CH
GR = CH // 16
NR = DIM // 16 + 1
ZR = 125
RPT = N // NS


def _fc_body(h_ref, wt_ref, a_ref, z_ref, pq_ref):
    z = h_ref[...] @ wt_ref[...]
    blk = z.shape[0]
    z_ref[...] = jnp.concatenate(
        [z, jnp.ones((blk, 1), jnp.float32), jnp.zeros((blk, WID - DIM - 1), jnp.float32)],
        axis=1)
    pq_ref[...] = z @ a_ref[...]


def _fc_call(h, W_fcT, A_pad):
    return pl.pallas_call(
        _fc_body,
        grid=(N // ROW_BLK,),
        in_specs=[
            pl.BlockSpec((ROW_BLK, DIM), lambda i: (i, 0)),
            pl.BlockSpec((DIM, DIM), lambda i: (0, 0)),
            pl.BlockSpec((DIM, DIM), lambda i: (0, 0)),
        ],
        out_specs=[
            pl.BlockSpec((ROW_BLK, WID), lambda i: (i, 0)),
            pl.BlockSpec((ROW_BLK, DIM), lambda i: (i, 0)),
        ],
        out_shape=[
            jax.ShapeDtypeStruct((N, WID), jnp.float32),
            jax.ShapeDtypeStruct((N, DIM), jnp.float32),
        ],
    )(h, W_fcT, A_pad)


SUP = 5
CPS = 25
RING = 3


def _edge_body(z_ref, src_ref, dst_ref, p_ref, q_ref, out_ref,
               src_blk, dst_blk, pg_v, qg_v, rows_v, out_sh,
               sem_g, sem_s):
    cid = lax.axis_index("c")
    sid = lax.axis_index("s")
    wid = sid * NC + cid

    def _zr(i, carry):
        r = i // NR
        j = i % NR
        rows_v[0][r, pl.ds(j * 16, 16)] = jnp.zeros((16,), jnp.float32)
        return carry
    lax.fori_loop(0, CH * NR, _zr, 0)
    base = sid * RPT
    for cz in range(RPT // CH):
        pltpu.sync_copy(rows_v[0], out_sh.at[pl.ds(base + cz * CH, CH)])
    pltpu.sync_copy(rows_v[0].at[pl.ds(0, RPT % CH)],
                    out_sh.at[pl.ds(base + (RPT // CH) * CH, RPT % CH)])
    plsc.subcore_barrier()

    sbase = wid * (EPW // CH)

    def _fire_gather(b, c):
        pltpu.async_copy(z_ref.at[src_blk.at[c]], rows_v[b], sem_g[b])
        pltpu.async_copy(p_ref.at[src_blk.at[c]], pg_v[b], sem_g[b])
        pltpu.async_copy(q_ref.at[dst_blk.at[c]], qg_v[b], sem_g[b])

    def _wait_gather(b):
        pltpu.make_async_copy(z_ref.at[src_blk.at[0]], rows_v[b], sem_g[b]).wait()
        pltpu.make_async_copy(p_ref.at[src_blk.at[0]], pg_v[b], sem_g[b]).wait()
        pltpu.make_async_copy(q_ref.at[dst_blk.at[0]], qg_v[b], sem_g[b]).wait()

    def _fire_scatter(b, c):
        pltpu.async_copy(rows_v[b], out_sh.at[dst_blk.at[c]], sem_s[b], add=True)

    def _wait_scatter(b):
        pltpu.make_async_copy(rows_v[b], out_sh.at[dst_blk.at[0]], sem_s[b]).wait()

    def _compute(b):
        def _g(g, carry):
            e = pg_v[b][pl.ds(g * 16, 16)] + qg_v[b][pl.ds(g * 16, 16)]
            e = jnp.where(e > 0.0, e, 0.01 * e)
            ex = jnp.exp(e)
            for kk in range(16):
                av = lax.gather(
                    ex, jnp.full((16, 1), kk, jnp.int32),
                    lax.GatherDimensionNumbers(
                        offset_dims=(), collapsed_slice_dims=(0,),
                        start_index_map=(0,)),
                    slice_sizes=(1,),
                    mode=lax.GatherScatterMode.PROMISE_IN_BOUNDS)
                row = g * 16 + kk
                for j in range(NR):
                    rows_v[b][row, pl.ds(j * 16, 16)] = (
                        rows_v[b][row, pl.ds(j * 16, 16)] * av)
            return carry
        lax.fori_loop(0, GR, _g, 0)

    def _super(s, carry):
        pltpu.sync_copy(src_ref.at[pl.ds(sbase + s * CPS, CPS)], src_blk)
        pltpu.sync_copy(dst_ref.at[pl.ds(sbase + s * CPS, CPS)], dst_blk)
        _fire_gather(0, 0)
        _fire_gather(1, 1)

        def _round(rr, carry2):
            for b in range(RING):
                c = rr * 3 + b
                _wait_gather(b)
                _compute(b)
                _fire_scatter(b, c)
                nb = (b + 2) % RING
                @pl.when(c + 2 < CPS)
                def _prefetch():
                    @pl.when(c >= 1)
                    def _drain():
                        _wait_scatter(nb)
                    _fire_gather(nb, c + 2)
            return carry2
        lax.fori_loop(0, CPS // RING, _round, 0)

        b_last = (CPS - 1) % RING
        _wait_gather(b_last)
        _compute(b_last)
        _fire_scatter(b_last, CPS - 1)
        for b in range(RING):
            _wait_scatter(b)
        return carry

    lax.fori_loop(0, SUP, _super, 0)

    plsc.subcore_barrier()
    pltpu.sync_copy(out_sh.at[pl.ds(sid * RPT, RPT)],
                    out_ref.at[cid, pl.ds(sid * RPT, RPT)])


@functools.partial(
    pl.kernel,
    out_type=jax.ShapeDtypeStruct((NC, N, WID), jnp.float32),
    mesh=plsc.VectorSubcoreMesh(
        core_axis_name="c", subcore_axis_name="s", num_cores=NC, num_subcores=NS),
    scratch_types=[
        pltpu.VMEM((CPS, CH), jnp.int32),
        pltpu.VMEM((CPS, CH), jnp.int32),
        [pltpu.VMEM((CH,), jnp.float32) for _ in range(RING)],
        [pltpu.VMEM((CH,), jnp.float32) for _ in range(RING)],
        [pltpu.VMEM((CH, WID), jnp.float32) for _ in range(RING)],
        pltpu.VMEM_SHARED((N, WID), jnp.float32),
        [pltpu.SemaphoreType.DMA for _ in range(RING)],
        [pltpu.SemaphoreType.DMA for _ in range(RING)],
    ],
    compiler_params=pltpu.CompilerParams(
        needs_layout_passes=False, use_tc_tiling_on_sc=False),
)
def _edge_call(z_ref, src_ref, dst_ref, p_ref, q_ref, out_ref,
               src_blk, dst_blk, pg_v, qg_v, rows_v, out_sh,
               sem_g, sem_s):
    _edge_body(z_ref, src_ref, dst_ref, p_ref, q_ref, out_ref,
               src_blk, dst_blk, pg_v, qg_v, rows_v, out_sh,
               sem_g, sem_s)


def _merge_body(parts_ref, out_ref):
    v = parts_ref[...]
    num = v[0, :, :DIM] + v[1, :, :DIM]
    den = v[0, :, DIM:DIM + 1] + v[1, :, DIM:DIM + 1]
    hval = num / jnp.maximum(den, 1e-16)
    out_ref[...] = jnp.where(hval > 0.0, hval, jnp.exp(hval) - 1.0)


def _merge_call(parts):
    return pl.pallas_call(
        _merge_body,
        grid=(N // ROW_BLK,),
        in_specs=[pl.BlockSpec((NC, ROW_BLK, WID), lambda i: (0, i, 0))],
        out_specs=pl.BlockSpec((ROW_BLK, DIM), lambda i: (i, 0)),
        out_shape=jax.ShapeDtypeStruct((N, DIM), jnp.float32),
    )(parts)


def kernel(h, edge_index, W_fc, W_attn):
    a_src = W_attn[0, :DIM]
    a_dst = W_attn[0, DIM:]
    A_pad = jnp.zeros((DIM, DIM), jnp.float32)
    A_pad = A_pad.at[:, 0].set(a_src).at[:, 1].set(a_dst)

    z_aug, pq = _fc_call(h, W_fc.T, A_pad)
    p = pq[:, 0]
    q = pq[:, 1]
    src = edge_index[0].reshape(E // CH, CH)
    dst = edge_index[1].reshape(E // CH, CH)

    parts = _edge_call(z_aug, src, dst, p, q)
    return _merge_call(parts)


# speedup vs baseline: 29.6124x; 1.0040x over previous
import functools

import jax
import jax.numpy as jnp
from jax import lax
from jax.experimental import pallas as pl
from jax.experimental.pallas import tpu as pltpu
from jax.experimental.pallas import tpu_sc as plsc

N = 10000
E = 320000
DIM = 128
WID = 144
ROW_BLK = 1000

NC = 2
NS = 16
NW = NC * NS
EPW = E // NW
CH = 80
NCH = EPW // CH
GR = CH // 16
NR = DIM // 16 + 1
ZR = 125
RPT = N // NS


def _fc_body(h_ref, wt_ref, a_ref, z_ref, pq_ref):
    z = h_ref[...] @ wt_ref[...]
    blk = z.shape[0]
    z_ref[...] = jnp.concatenate(
        [z, jnp.ones((blk, 1), jnp.float32), jnp.zeros((blk, WID - DIM - 1), jnp.float32)],
        axis=1)
    pq_ref[...] = z @ a_ref[...]


def _fc_call(h, W_fcT, A_pad):
    return pl.pallas_call(
        _fc_body,
        grid=(N // ROW_BLK,),
        in_specs=[
            pl.BlockSpec((ROW_BLK, DIM), lambda i: (i, 0)),
            pl.BlockSpec((DIM, DIM), lambda i: (0, 0)),
            pl.BlockSpec((DIM, DIM), lambda i: (0, 0)),
        ],
        out_specs=[
            pl.BlockSpec((ROW_BLK, WID), lambda i: (i, 0)),
            pl.BlockSpec((ROW_BLK, DIM), lambda i: (i, 0)),
        ],
        out_shape=[
            jax.ShapeDtypeStruct((N, WID), jnp.float32),
            jax.ShapeDtypeStruct((N, DIM), jnp.float32),
        ],
    )(h, W_fcT, A_pad)


SUP = 5
CPS = 25
RING = 3


def _edge_body(z_ref, src_ref, dst_ref, p_ref, q_ref, out_ref,
               src_blk, dst_blk, pg_v, qg_v, rows_v, out_sh,
               sem_g, sem_s):
    cid = lax.axis_index("c")
    sid = lax.axis_index("s")
    wid = sid * NC + cid

    def _zr(i, carry):
        r = i // NR
        j = i % NR
        rows_v[0][r, pl.ds(j * 16, 16)] = jnp.zeros((16,), jnp.float32)
        return carry
    lax.fori_loop(0, CH * NR, _zr, 0)
    base = sid * RPT
    for cz in range(RPT // CH):
        pltpu.async_copy(rows_v[0], out_sh.at[pl.ds(base + cz * CH, CH)], sem_s[0])
    pltpu.async_copy(rows_v[0].at[pl.ds(0, RPT % CH)],
                     out_sh.at[pl.ds(base + (RPT // CH) * CH, RPT % CH)], sem_s[0])
    for cz in range(RPT // CH):
        pltpu.make_async_copy(rows_v[0], out_sh.at[pl.ds(base, CH)], sem_s[0]).wait()
    pltpu.make_async_copy(rows_v[0].at[pl.ds(0, RPT % CH)],
                          out_sh.at[pl.ds(base, RPT % CH)], sem_s[0]).wait()
    plsc.subcore_barrier()

    sbase = wid * (EPW // CH)

    def _fire_gather(b, c):
        pltpu.async_copy(z_ref.at[src_blk.at[c]], rows_v[b], sem_g[b])
        pltpu.async_copy(p_ref.at[src_blk.at[c]], pg_v[b], sem_g[b])
        pltpu.async_copy(q_ref.at[dst_blk.at[c]], qg_v[b], sem_g[b])

    def _wait_gather(b):
        pltpu.make_async_copy(z_ref.at[src_blk.at[0]], rows_v[b], sem_g[b]).wait()
        pltpu.make_async_copy(p_ref.at[src_blk.at[0]], pg_v[b], sem_g[b]).wait()
        pltpu.make_async_copy(q_ref.at[dst_blk.at[0]], qg_v[b], sem_g[b]).wait()

    def _fire_scatter(b, c):
        pltpu.async_copy(rows_v[b], out_sh.at[dst_blk.at[c]], sem_s[b], add=True)

    def _wait_scatter(b):
        pltpu.make_async_copy(rows_v[b], out_sh.at[dst_blk.at[0]], sem_s[b]).wait()

    def _compute(b):
        def _g(g, carry):
            e = pg_v[b][pl.ds(g * 16, 16)] + qg_v[b][pl.ds(g * 16, 16)]
            e = jnp.where(e > 0.0, e, 0.01 * e)
            ex = jnp.exp(e)
            for kk in range(16):
                av = lax.gather(
                    ex, jnp.full((16, 1), kk, jnp.int32),
                    lax.GatherDimensionNumbers(
                        offset_dims=(), collapsed_slice_dims=(0,),
                        start_index_map=(0,)),
                    slice_sizes=(1,),
                    mode=lax.GatherScatterMode.PROMISE_IN_BOUNDS)
                row = g * 16 + kk
                for j in range(NR):
                    rows_v[b][row, pl.ds(j * 16, 16)] = (
                        rows_v[b][row, pl.ds(j * 16, 16)] * av)
            return carry
        lax.fori_loop(0, GR, _g, 0)

    def _super(s, carry):
        pltpu.sync_copy(src_ref.at[pl.ds(sbase + s * CPS, CPS)], src_blk)
        pltpu.sync_copy(dst_ref.at[pl.ds(sbase + s * CPS, CPS)], dst_blk)
        _fire_gather(0, 0)
        _fire_gather(1, 1)

        def _round(rr, carry2):
            for b in range(RING):
                c = rr * 3 + b
                _wait_gather(b)
                _compute(b)
                _fire_scatter(b, c)
                nb = (b + 2) % RING
                @pl.when(c + 2 < CPS)
                def _prefetch():
                    @pl.when(c >= 1)
                    def _drain():
                        _wait_scatter(nb)
                    _fire_gather(nb, c + 2)
            return carry2
        lax.fori_loop(0, CPS // RING, _round, 0)

        b_last = (CPS - 1) % RING
        _wait_gather(b_last)
        _compute(b_last)
        _fire_scatter(b_last, CPS - 1)
        for b in range(RING):
            _wait_scatter(b)
        return carry

    lax.fori_loop(0, SUP, _super, 0)

    plsc.subcore_barrier()
    pltpu.sync_copy(out_sh.at[pl.ds(sid * RPT, RPT)],
                    out_ref.at[cid, pl.ds(sid * RPT, RPT)])


@functools.partial(
    pl.kernel,
    out_type=jax.ShapeDtypeStruct((NC, N, WID), jnp.float32),
    mesh=plsc.VectorSubcoreMesh(
        core_axis_name="c", subcore_axis_name="s", num_cores=NC, num_subcores=NS),
    scratch_types=[
        pltpu.VMEM((CPS, CH), jnp.int32),
        pltpu.VMEM((CPS, CH), jnp.int32),
        [pltpu.VMEM((CH,), jnp.float32) for _ in range(RING)],
        [pltpu.VMEM((CH,), jnp.float32) for _ in range(RING)],
        [pltpu.VMEM((CH, WID), jnp.float32) for _ in range(RING)],
        pltpu.VMEM_SHARED((N, WID), jnp.float32),
        [pltpu.SemaphoreType.DMA for _ in range(RING)],
        [pltpu.SemaphoreType.DMA for _ in range(RING)],
    ],
    compiler_params=pltpu.CompilerParams(
        needs_layout_passes=False, use_tc_tiling_on_sc=False),
)
def _edge_call(z_ref, src_ref, dst_ref, p_ref, q_ref, out_ref,
               src_blk, dst_blk, pg_v, qg_v, rows_v, out_sh,
               sem_g, sem_s):
    _edge_body(z_ref, src_ref, dst_ref, p_ref, q_ref, out_ref,
               src_blk, dst_blk, pg_v, qg_v, rows_v, out_sh,
               sem_g, sem_s)


def _merge_body(parts_ref, out_ref):
    v = parts_ref[...]
    num = v[0, :, :DIM] + v[1, :, :DIM]
    den = v[0, :, DIM:DIM + 1] + v[1, :, DIM:DIM + 1]
    hval = num / jnp.maximum(den, 1e-16)
    out_ref[...] = jnp.where(hval > 0.0, hval, jnp.exp(hval) - 1.0)


def _merge_call(parts):
    return pl.pallas_call(
        _merge_body,
        grid=(N // ROW_BLK,),
        in_specs=[pl.BlockSpec((NC, ROW_BLK, WID), lambda i: (0, i, 0))],
        out_specs=pl.BlockSpec((ROW_BLK, DIM), lambda i: (i, 0)),
        out_shape=jax.ShapeDtypeStruct((N, DIM), jnp.float32),
    )(parts)


def kernel(h, edge_index, W_fc, W_attn):
    a_src = W_attn[0, :DIM]
    a_dst = W_attn[0, DIM:]
    A_pad = jnp.zeros((DIM, DIM), jnp.float32)
    A_pad = A_pad.at[:, 0].set(a_src).at[:, 1].set(a_dst)

    z_aug, pq = _fc_call(h, W_fc.T, A_pad)
    p = pq[:, 0]
    q = pq[:, 1]
    src = edge_index[0].reshape(E // CH, CH)
    dst = edge_index[1].reshape(E // CH, CH)

    parts = _edge_call(z_aug, src, dst, p, q)
    return _merge_call(parts)
